# baseline (speedup 1.0000x reference)
import functools

import jax
import jax.numpy as jnp
from jax import lax
from jax.experimental import pallas as pl
from jax.experimental.pallas import tpu as pltpu
from jax.experimental.pallas import tpu_sc as plsc

_B, _N, _CIN = 8, 16384, 32
_C1, _C2 = 32, 64
_NPOINT = 512
_M = _B * _N
_CHUNK = 8192
_GRID = _M // _CHUNK
_EPS = 1e-5


def _stats0_body(x_ref, g_ref, s_ref):
    step = pl.program_id(0)
    x = x_ref[...]

    @pl.when(step == 0)
    def _():
        g_ref[...] = jnp.zeros_like(g_ref)
        s_ref[...] = jnp.zeros_like(s_ref)

    g_ref[...] += jax.lax.dot_general(
        x, x, (((0,), (0,)), ((), ())), preferred_element_type=jnp.float32)
    s_ref[...] += jnp.sum(x, axis=0, keepdims=True)


def _fold_layer(W, b, g, bb, mean_in, gram_in):
    cov = gram_in - jax.lax.dot_general(
        mean_in, mean_in, (((0,), (0,)), ((), ())),
        preferred_element_type=jnp.float32)
    a = jax.lax.dot_general(W, cov, (((1,), (0,)), ((), ())),
                            preferred_element_type=jnp.float32)
    a = jax.lax.dot_general(a, W, (((1,), (1,)), ((), ())),
                            preferred_element_type=jnp.float32)
    c = W.shape[0]
    eye = (lax.broadcasted_iota(jnp.int32, (c, c), 0)
           == lax.broadcasted_iota(jnp.int32, (c, c), 1)).astype(jnp.float32)
    var = jnp.sum(a * eye, axis=1)
    mean = jnp.dot(mean_in, W.T, preferred_element_type=jnp.float32)[0] + b
    scale = g * lax.rsqrt(var + _EPS)
    Wp = W * scale[:, None]
    cp = scale * (b - mean) + bb
    return Wp, cp


def _zstats_body(x_ref, g_ref, s_ref, w0_ref, b0_ref, g0_ref, bb0_ref,
                 gz_ref, sz_ref):
    step = pl.program_id(0)
    w0p, c0p = _fold_layer(w0_ref[...], b0_ref[...], g0_ref[...], bb0_ref[...],
                           s_ref[...] / _M, g_ref[...] / _M)
    x = x_ref[...]
    z = jnp.maximum(
        jax.lax.dot_general(x, w0p, (((1,), (1,)), ((), ())),
                            preferred_element_type=jnp.float32) + c0p, 0.0)

    @pl.when(step == 0)
    def _():
        gz_ref[...] = jnp.zeros_like(gz_ref)
        sz_ref[...] = jnp.zeros_like(sz_ref)

    gz_ref[...] += jax.lax.dot_general(
        z, z, (((0,), (0,)), ((), ())), preferred_element_type=jnp.float32)
    sz_ref[...] += jnp.sum(z, axis=0, keepdims=True)


def _final_body(x_ref, g_ref, s_ref, gz_ref, sz_ref,
                w0_ref, b0_ref, g0_ref, bb0_ref,
                w1_ref, b1_ref, g1_ref, bb1_ref, out_ref):
    step = pl.program_id(0)
    w0p, c0p = _fold_layer(w0_ref[...], b0_ref[...], g0_ref[...], bb0_ref[...],
                           s_ref[...] / _M, g_ref[...] / _M)
    w1p, c1p = _fold_layer(w1_ref[...], b1_ref[...], g1_ref[...], bb1_ref[...],
                           sz_ref[...] / _M, gz_ref[...] / _M)
    x = x_ref[...]
    z = jnp.maximum(
        jax.lax.dot_general(x, w0p, (((1,), (1,)), ((), ())),
                            preferred_element_type=jnp.float32) + c0p, 0.0)
    h = jnp.maximum(
        jax.lax.dot_general(z, w1p, (((1,), (1,)), ((), ())),
                            preferred_element_type=jnp.float32) + c1p, 0.0)

    @pl.when(step % (_N // _CHUNK) == 0)
    def _():
        out_ref[...] = jnp.zeros_like(out_ref)

    out_ref[...] += jnp.sum(h, axis=0, keepdims=True)[None] * (1.0 / _N)


def _mlp(points, w0, b0, g0, bb0, w1, b1, g1, bb1):
    x = points.reshape(_M, _CIN)
    full = lambda shape: pl.BlockSpec(shape, lambda i: (0,) * len(shape))
    xspec = pl.BlockSpec((_CHUNK, _CIN), lambda i: (i, 0))

    g, s = pl.pallas_call(
        _stats0_body,
        grid=(_GRID,),
        in_specs=[xspec],
        out_specs=[full((_CIN, _CIN)), full((1, _CIN))],
        out_shape=[jax.ShapeDtypeStruct((_CIN, _CIN), jnp.float32),
                   jax.ShapeDtypeStruct((1, _CIN), jnp.float32)],
        compiler_params=pltpu.CompilerParams(
            dimension_semantics=("arbitrary",)),
    )(x)

    gz, sz = pl.pallas_call(
        _zstats_body,
        grid=(_GRID,),
        in_specs=[xspec, full((_CIN, _CIN)), full((1, _CIN)),
                  full((_C1, _CIN)), full((_C1,)), full((_C1,)), full((_C1,))],
        out_specs=[full((_C1, _C1)), full((1, _C1))],
        out_shape=[jax.ShapeDtypeStruct((_C1, _C1), jnp.float32),
                   jax.ShapeDtypeStruct((1, _C1), jnp.float32)],
        compiler_params=pltpu.CompilerParams(
            dimension_semantics=("arbitrary",)),
    )(x, g, s, w0, b0, g0, bb0)

    out = pl.pallas_call(
        _final_body,
        grid=(_GRID,),
        in_specs=[xspec, full((_CIN, _CIN)), full((1, _CIN)),
                  full((_C1, _C1)), full((1, _C1)),
                  full((_C1, _CIN)), full((_C1,)), full((_C1,)), full((_C1,)),
                  full((_C2, _C1)), full((_C2,)), full((_C2,)), full((_C2,))],
        out_specs=pl.BlockSpec((1, 1, _C2), lambda i: (i // (_N // _CHUNK), 0, 0)),
        out_shape=jax.ShapeDtypeStruct((_B, 1, _C2), jnp.float32),
        compiler_params=pltpu.CompilerParams(
            dimension_semantics=("arbitrary",)),
    )(x, g, s, gz, sz, w0, b0, g0, bb0, w1, b1, g1, bb1)

    return jnp.transpose(out, (0, 2, 1))


_GSH = 8
_PSH = _N // _GSH
_UNROLL = 8
_BIG = 3.0e38
_CSLOT = _GSH * 64
_PSTRIDE = 4 * _CSLOT


def _fps_sc_body(xyz_hbm, out_hbm, xsA, ysA, zsA, dA, xsB, ysB, zsB, dB,
                 pub, candA, candB, nxA, nxB, semA, semB, shared):
    c = lax.axis_index("c")
    s = lax.axis_index("s")
    g = s // _GSH
    shard = s % _GSH
    clA = g * 2
    clB = g * 2 + 1
    bA = c * 4 + clA
    bB = c * 4 + clB
    base = shard * _PSH

    for bb, xs_, ys_, zs_ in ((bA, xsA, ysA, zsA), (bB, xsB, ysB, zsB)):
        src = bb * (3 * _N) + base
        pltpu.sync_copy(xyz_hbm.at[pl.ds(src, _PSH)], xs_)
        pltpu.sync_copy(xyz_hbm.at[pl.ds(src + _N, _PSH)], ys_)
        pltpu.sync_copy(xyz_hbm.at[pl.ds(src + 2 * _N, _PSH)], zs_)

    lanes = lax.iota(jnp.int32, 16)
    zero16 = jnp.zeros((16,), jnp.int32)

    def _init(i, carry):
        dA[pl.ds(i * 16, 16)] = jnp.full((16,), 1e10, jnp.float32)
        dB[pl.ds(i * 16, 16)] = jnp.full((16,), 1e10, jnp.float32)
        return carry

    lax.fori_loop(0, _PSH // 16, _init, 0)

    def lane_bcast(vec, lane_idx):
        sel = lanes == jnp.full((16,), lane_idx, jnp.int32)
        picked = jnp.where(sel, vec, jnp.full((16,), -_BIG, jnp.float32))
        return jnp.full((16,), jnp.max(picked), jnp.float32)

    def cand_slice(cl, parity):
        return shared.at[pl.ds(parity * _PSTRIDE + cl * _CSLOT, _CSLOT)]

    is0 = jnp.full((16,), shard, jnp.int32) == zero16

    def publish_initial(xs_, ys_, zs_, cl):
        pub[pl.ds(0, 16)] = jnp.where(is0, _BIG, -_BIG)
        pub[pl.ds(16, 16)] = jnp.where(is0, lane_bcast(xs_[pl.ds(0, 16)], 0),
                                       0.0)
        pub[pl.ds(32, 16)] = jnp.where(is0, lane_bcast(ys_[pl.ds(0, 16)], 0),
                                       0.0)
        pub[pl.ds(48, 16)] = jnp.where(is0, lane_bcast(zs_[pl.ds(0, 16)], 0),
                                       0.0)
        pltpu.sync_copy(pub, shared.at[pl.ds(cl * _CSLOT + shard * 64, 64)])

    publish_initial(xsA, ysA, zsA, clA)
    publish_initial(xsB, ysB, zsB, clB)
    plsc.subcore_barrier()
    pltpu.async_copy(cand_slice(clA, 0), candA, semA)
    pltpu.async_copy(cand_slice(clB, 0), candB, semB)

    def phase(t, parity, xs_, ys_, zs_, d_, cand_, sem, nx_, cl):
        pltpu.make_async_copy(cand_slice(cl, parity), cand_, sem).wait()
        row = lambda k, comp: cand_[pl.ds(k * 64 + comp * 16, 16)]
        wm = row(0, 0)
        wx = row(0, 1)
        wy = row(0, 2)
        wz = row(0, 3)
        for k in range(1, _GSH):
            better = row(k, 0) > wm
            wm = jnp.where(better, row(k, 0), wm)
            wx = jnp.where(better, row(k, 1), wx)
            wy = jnp.where(better, row(k, 2), wy)
            wz = jnp.where(better, row(k, 3), wz)
        pack = jnp.where(lanes == 1, wx,
                         jnp.where(lanes == 2, wy,
                                   jnp.where(lanes == 3, wz, wm)))
        nx_[pl.ds(t * 16, 16)] = pack

        @plsc.parallel_loop(0, _PSH, 16, unroll=_UNROLL,
                            carry=(jnp.full((16,), -_BIG, jnp.float32),
                                   zero16))
        def dloop(off, mc):
            m, idxv = mc
            xv = xs_[pl.ds(off, 16)]
            yv = ys_[pl.ds(off, 16)]
            zv = zs_[pl.ds(off, 16)]
            dx = xv - wx
            dy = yv - wy
            dz = zv - wz
            d = dx * dx + dy * dy
            d = d + dz * dz
            nd = jnp.minimum(d_[pl.ds(off, 16)], d)
            d_[pl.ds(off, 16)] = nd
            upd = nd > m
            m = jnp.where(upd, nd, m)
            idxv = jnp.where(upd, lanes + off, idxv)
            return m, idxv

        m, idxv = dloop

        lmv = jnp.full((16,), jnp.max(m), jnp.float32)
        iv = jnp.where(m == lmv, idxv, jnp.full((16,), 1 << 30, jnp.int32))
        li = jnp.min(iv)
        blk = (li // 16) * 16
        lane_idx = li - blk
        pub[pl.ds(0, 16)] = lmv
        pub[pl.ds(16, 16)] = lane_bcast(xs_[pl.ds(blk, 16)], lane_idx)
        pub[pl.ds(32, 16)] = lane_bcast(ys_[pl.ds(blk, 16)], lane_idx)
        pub[pl.ds(48, 16)] = lane_bcast(zs_[pl.ds(blk, 16)], lane_idx)
        pltpu.sync_copy(
            pub, shared.at[pl.ds((1 - parity) * _PSTRIDE + cl * _CSLOT
                                 + shard * 64, 64)])
        plsc.subcore_barrier()
        pltpu.async_copy(cand_slice(cl, 1 - parity), cand_, sem)

    def one_step(t, parity):
        phase(t, parity, xsA, ysA, zsA, dA, candA, semA, nxA, clA)
        phase(t, parity, xsB, ysB, zsB, dB, candB, semB, nxB, clB)

    def outer(i, carry):
        one_step(2 * i, 0)
        one_step(2 * i + 1, 1)
        return carry

    lax.fori_loop(0, _NPOINT // 2, outer, 0)

    pltpu.make_async_copy(cand_slice(clA, 0), candA, semA).wait()
    pltpu.make_async_copy(cand_slice(clB, 0), candB, semB).wait()

    @pl.when(shard == 0)
    def _():
        pltpu.sync_copy(nxA, out_hbm.at[pl.ds(bA * (_NPOINT * 16),
                                              _NPOINT * 16)])

    @pl.when(shard == 1)
    def _():
        pltpu.sync_copy(nxB, out_hbm.at[pl.ds(bB * (_NPOINT * 16),
                                              _NPOINT * 16)])


def _fps_new_xyz(xyz):
    xyz_t = jnp.transpose(xyz, (0, 2, 1)).reshape(_B * 3 * _N)
    mesh = plsc.VectorSubcoreMesh(core_axis_name="c", subcore_axis_name="s",
                                  num_cores=2, num_subcores=16)
    out = pl.kernel(
        _fps_sc_body,
        out_type=jax.ShapeDtypeStruct((_B * _NPOINT * 16,), jnp.float32),
        mesh=mesh,
        compiler_params=pltpu.CompilerParams(needs_layout_passes=False),
        scratch_types=[
            pltpu.VMEM((_PSH,), jnp.float32),
            pltpu.VMEM((_PSH,), jnp.float32),
            pltpu.VMEM((_PSH,), jnp.float32),
            pltpu.VMEM((_PSH,), jnp.float32),
            pltpu.VMEM((_PSH,), jnp.float32),
            pltpu.VMEM((_PSH,), jnp.float32),
            pltpu.VMEM((_PSH,), jnp.float32),
            pltpu.VMEM((_PSH,), jnp.float32),
            pltpu.VMEM((64,), jnp.float32),
            pltpu.VMEM((_CSLOT,), jnp.float32),
            pltpu.VMEM((_CSLOT,), jnp.float32),
            pltpu.VMEM((_NPOINT * 16,), jnp.float32),
            pltpu.VMEM((_NPOINT * 16,), jnp.float32),
            pltpu.SemaphoreType.DMA,
            pltpu.SemaphoreType.DMA,
            pltpu.VMEM_SHARED((2 * _PSTRIDE,), jnp.float32),
        ],
    )(xyz_t)
    return out.reshape(_B, _NPOINT, 16)[:, :, 1:4]


def kernel(xyz, points, conv_w0, conv_b0, bn_g0, bn_b0,
           conv_w1, conv_b1, bn_g1, bn_b1):
    new_xyz = _fps_new_xyz(xyz)
    trans_feat = _mlp(points, conv_w0, conv_b0, bn_g0, bn_b0,
                      conv_w1, conv_b1, bn_g1, bn_b1)
    return (new_xyz, trans_feat)


# speedup vs baseline: 1.0917x; 1.0917x over previous
import jax
import jax.numpy as jnp
from jax import lax
from jax.experimental import pallas as pl
from jax.experimental.pallas import tpu as pltpu
from jax.experimental.pallas import tpu_sc as plsc

_B, _N, _CIN = 8, 16384, 32
_C1, _C2 = 32, 64
_NPOINT = 512
_M = _B * _N
_CHUNK = 8192
_GRID = _M // _CHUNK
_EPS = 1e-5


def _stats0_body(x_ref, g_ref, s_ref):
    step = pl.program_id(0)
    x = x_ref[...]

    @pl.when(step == 0)
    def _():
        g_ref[...] = jnp.zeros_like(g_ref)
        s_ref[...] = jnp.zeros_like(s_ref)

    g_ref[...] += jax.lax.dot_general(
        x, x, (((0,), (0,)), ((), ())), preferred_element_type=jnp.float32)
    s_ref[...] += jnp.sum(x, axis=0, keepdims=True)


def _fold_layer(W, b, g, bb, mean_in, gram_in):
    cov = gram_in - jax.lax.dot_general(
        mean_in, mean_in, (((0,), (0,)), ((), ())),
        preferred_element_type=jnp.float32)
    a = jax.lax.dot_general(W, cov, (((1,), (0,)), ((), ())),
                            preferred_element_type=jnp.float32)
    a = jax.lax.dot_general(a, W, (((1,), (1,)), ((), ())),
                            preferred_element_type=jnp.float32)
    c = W.shape[0]
    eye = (lax.broadcasted_iota(jnp.int32, (c, c), 0)
           == lax.broadcasted_iota(jnp.int32, (c, c), 1)).astype(jnp.float32)
    var = jnp.sum(a * eye, axis=1)
    mean = jnp.dot(mean_in, W.T, preferred_element_type=jnp.float32)[0] + b
    scale = g * lax.rsqrt(var + _EPS)
    Wp = W * scale[:, None]
    cp = scale * (b - mean) + bb
    return Wp, cp


def _zstats_body(x_ref, g_ref, s_ref, w0_ref, b0_ref, g0_ref, bb0_ref,
                 gz_ref, sz_ref):
    step = pl.program_id(0)
    w0p, c0p = _fold_layer(w0_ref[...], b0_ref[...], g0_ref[...], bb0_ref[...],
                           s_ref[...] / _M, g_ref[...] / _M)
    x = x_ref[...]
    z = jnp.maximum(
        jax.lax.dot_general(x, w0p, (((1,), (1,)), ((), ())),
                            preferred_element_type=jnp.float32) + c0p, 0.0)

    @pl.when(step == 0)
    def _():
        gz_ref[...] = jnp.zeros_like(gz_ref)
        sz_ref[...] = jnp.zeros_like(sz_ref)

    gz_ref[...] += jax.lax.dot_general(
        z, z, (((0,), (0,)), ((), ())), preferred_element_type=jnp.float32)
    sz_ref[...] += jnp.sum(z, axis=0, keepdims=True)


def _final_body(x_ref, g_ref, s_ref, gz_ref, sz_ref,
                w0_ref, b0_ref, g0_ref, bb0_ref,
                w1_ref, b1_ref, g1_ref, bb1_ref, out_ref):
    step = pl.program_id(0)
    w0p, c0p = _fold_layer(w0_ref[...], b0_ref[...], g0_ref[...], bb0_ref[...],
                           s_ref[...] / _M, g_ref[...] / _M)
    w1p, c1p = _fold_layer(w1_ref[...], b1_ref[...], g1_ref[...], bb1_ref[...],
                           sz_ref[...] / _M, gz_ref[...] / _M)
    x = x_ref[...]
    z = jnp.maximum(
        jax.lax.dot_general(x, w0p, (((1,), (1,)), ((), ())),
                            preferred_element_type=jnp.float32) + c0p, 0.0)
    h = jnp.maximum(
        jax.lax.dot_general(z, w1p, (((1,), (1,)), ((), ())),
                            preferred_element_type=jnp.float32) + c1p, 0.0)

    @pl.when(step % (_N // _CHUNK) == 0)
    def _():
        out_ref[...] = jnp.zeros_like(out_ref)

    out_ref[...] += jnp.sum(h, axis=0, keepdims=True)[None] * (1.0 / _N)


def _mlp(points, w0, b0, g0, bb0, w1, b1, g1, bb1):
    x = points.reshape(_M, _CIN)
    full = lambda shape: pl.BlockSpec(shape, lambda i: (0,) * len(shape))
    xspec = pl.BlockSpec((_CHUNK, _CIN), lambda i: (i, 0))

    g, s = pl.pallas_call(
        _stats0_body,
        grid=(_GRID,),
        in_specs=[xspec],
        out_specs=[full((_CIN, _CIN)), full((1, _CIN))],
        out_shape=[jax.ShapeDtypeStruct((_CIN, _CIN), jnp.float32),
                   jax.ShapeDtypeStruct((1, _CIN), jnp.float32)],
        compiler_params=pltpu.CompilerParams(
            dimension_semantics=("arbitrary",)),
    )(x)

    gz, sz = pl.pallas_call(
        _zstats_body,
        grid=(_GRID,),
        in_specs=[xspec, full((_CIN, _CIN)), full((1, _CIN)),
                  full((_C1, _CIN)), full((_C1,)), full((_C1,)), full((_C1,))],
        out_specs=[full((_C1, _C1)), full((1, _C1))],
        out_shape=[jax.ShapeDtypeStruct((_C1, _C1), jnp.float32),
                   jax.ShapeDtypeStruct((1, _C1), jnp.float32)],
        compiler_params=pltpu.CompilerParams(
            dimension_semantics=("arbitrary",)),
    )(x, g, s, w0, b0, g0, bb0)

    out = pl.pallas_call(
        _final_body,
        grid=(_GRID,),
        in_specs=[xspec, full((_CIN, _CIN)), full((1, _CIN)),
                  full((_C1, _C1)), full((1, _C1)),
                  full((_C1, _CIN)), full((_C1,)), full((_C1,)), full((_C1,)),
                  full((_C2, _C1)), full((_C2,)), full((_C2,)), full((_C2,))],
        out_specs=pl.BlockSpec((1, 1, _C2), lambda i: (i // (_N // _CHUNK), 0, 0)),
        out_shape=jax.ShapeDtypeStruct((_B, 1, _C2), jnp.float32),
        compiler_params=pltpu.CompilerParams(
            dimension_semantics=("arbitrary",)),
    )(x, g, s, gz, sz, w0, b0, g0, bb0, w1, b1, g1, bb1)

    return jnp.transpose(out, (0, 2, 1))


_SHARDS = 4
_SH = _N // _SHARDS
_NV = _SH // 16
_UNROLL = 8
_BIG = 3.0e38


def _fps_sc_body(xyz_hbm, out_hbm, xs, ys, zs, dist, pub, cand, nxbuf, shared):
    c = lax.axis_index("c")
    s = lax.axis_index("s")
    b = c * 4 + s // _SHARDS
    bl = s // _SHARDS
    shard = s % _SHARDS
    base = shard * _SH

    src = b * (3 * _N) + base
    pltpu.sync_copy(xyz_hbm.at[pl.ds(src, _SH)], xs)
    pltpu.sync_copy(xyz_hbm.at[pl.ds(src + _N, _SH)], ys)
    pltpu.sync_copy(xyz_hbm.at[pl.ds(src + 2 * _N, _SH)], zs)

    lanes = lax.iota(jnp.int32, 16)
    zero16 = jnp.zeros((16,), jnp.int32)

    def _init(i, carry):
        dist[pl.ds(i * 16, 16)] = jnp.full((16,), 1e10, jnp.float32)
        return carry

    lax.fori_loop(0, _NV, _init, 0)

    def lane_bcast(vec, lane_idx):
        sel = lanes == jnp.full((16,), lane_idx, jnp.int32)
        picked = jnp.where(sel, vec, jnp.full((16,), -_BIG, jnp.float32))
        return jnp.full((16,), jnp.max(picked), jnp.float32)

    is0 = jnp.full((16,), shard, jnp.int32) == zero16
    pub[pl.ds(0, 16)] = jnp.where(is0, _BIG, -_BIG)
    pub[pl.ds(16, 16)] = jnp.where(is0, lane_bcast(xs[pl.ds(0, 16)], 0), 0.0)
    pub[pl.ds(32, 16)] = jnp.where(is0, lane_bcast(ys[pl.ds(0, 16)], 0), 0.0)
    pub[pl.ds(48, 16)] = jnp.where(is0, lane_bcast(zs[pl.ds(0, 16)], 0), 0.0)
    slot = (bl * _SHARDS + shard) * 64
    pltpu.sync_copy(pub, shared.at[pl.ds(slot, 64)])
    plsc.subcore_barrier()

    def one_step(t, parity):
        pltpu.sync_copy(
            shared.at[pl.ds(parity * (4 * _SHARDS * 64) + bl * (_SHARDS * 64),
                            _SHARDS * 64)], cand)
        row = lambda k, comp: cand[pl.ds(k * 64 + comp * 16, 16)]
        wm = row(0, 0)
        wx = row(0, 1)
        wy = row(0, 2)
        wz = row(0, 3)
        for k in range(1, _SHARDS):
            better = row(k, 0) > wm
            wm = jnp.where(better, row(k, 0), wm)
            wx = jnp.where(better, row(k, 1), wx)
            wy = jnp.where(better, row(k, 2), wy)
            wz = jnp.where(better, row(k, 3), wz)
        pack = jnp.where(lanes == 1, wx,
                         jnp.where(lanes == 2, wy,
                                   jnp.where(lanes == 3, wz, wm)))
        nxbuf[pl.ds(t * 16, 16)] = pack

        @plsc.parallel_loop(0, _SH, 16, unroll=_UNROLL,
                            carry=(jnp.full((16,), -_BIG, jnp.float32),
                                   zero16))
        def dloop(off, mc):
            m, idxv = mc
            xv = xs[pl.ds(off, 16)]
            yv = ys[pl.ds(off, 16)]
            zv = zs[pl.ds(off, 16)]
            dx = xv - wx
            dy = yv - wy
            dz = zv - wz
            d = dx * dx + dy * dy
            d = d + dz * dz
            nd = jnp.minimum(dist[pl.ds(off, 16)], d)
            dist[pl.ds(off, 16)] = nd
            upd = nd > m
            m = jnp.where(upd, nd, m)
            idxv = jnp.where(upd, lanes + off, idxv)
            return m, idxv

        m, idxv = dloop

        lmv = jnp.full((16,), jnp.max(m), jnp.float32)
        iv = jnp.where(m == lmv, idxv, jnp.full((16,), 1 << 30, jnp.int32))
        li = jnp.min(iv)
        blk = (li // 16) * 16
        lane_idx = li - blk
        pub[pl.ds(0, 16)] = lmv
        pub[pl.ds(16, 16)] = lane_bcast(xs[pl.ds(blk, 16)], lane_idx)
        pub[pl.ds(32, 16)] = lane_bcast(ys[pl.ds(blk, 16)], lane_idx)
        pub[pl.ds(48, 16)] = lane_bcast(zs[pl.ds(blk, 16)], lane_idx)
        pltpu.sync_copy(
            pub,
            shared.at[pl.ds((1 - parity) * (4 * _SHARDS * 64)
                            + (bl * _SHARDS + shard) * 64, 64)])
        plsc.subcore_barrier()

    def outer(i, carry):
        one_step(2 * i, 0)
        one_step(2 * i + 1, 1)
        return carry

    lax.fori_loop(0, _NPOINT // 2, outer, 0)

    @pl.when(shard == 0)
    def _():
        pltpu.sync_copy(nxbuf, out_hbm.at[pl.ds(b * (_NPOINT * 16),
                                                _NPOINT * 16)])


def _fps_new_xyz(xyz):
    xyz_t = jnp.transpose(xyz, (0, 2, 1)).reshape(_B * 3 * _N)
    mesh = plsc.VectorSubcoreMesh(core_axis_name="c", subcore_axis_name="s",
                                  num_cores=2, num_subcores=16)
    out = pl.kernel(
        _fps_sc_body,
        out_type=jax.ShapeDtypeStruct((_B * _NPOINT * 16,), jnp.float32),
        mesh=mesh,
        compiler_params=pltpu.CompilerParams(needs_layout_passes=False),
        scratch_types=[
            pltpu.VMEM((_SH,), jnp.float32),
            pltpu.VMEM((_SH,), jnp.float32),
            pltpu.VMEM((_SH,), jnp.float32),
            pltpu.VMEM((_SH,), jnp.float32),
            pltpu.VMEM((64,), jnp.float32),
            pltpu.VMEM((_SHARDS * 64,), jnp.float32),
            pltpu.VMEM((_NPOINT * 16,), jnp.float32),
            pltpu.VMEM_SHARED((2 * 4 * _SHARDS * 64,), jnp.float32),
        ],
    )(xyz_t)
    return out.reshape(_B, _NPOINT, 16)[:, :, 1:4]


def kernel(xyz, points, conv_w0, conv_b0, bn_g0, bn_b0,
           conv_w1, conv_b1, bn_g1, bn_b1):
    new_xyz = _fps_new_xyz(xyz)
    trans_feat = _mlp(points, conv_w0, conv_b0, bn_g0, bn_b0,
                      conv_w1, conv_b1, bn_g1, bn_b1)
    return (new_xyz, trans_feat)
